# Initial kernel scaffold; baseline (speedup 1.0000x reference)
import jax
import jax.numpy as jnp
from jax.experimental import pallas as pl


def kernel(x, edge_index, Wl1, Wr1, att1, b1, Wl2, Wr2, att2, b2, Wlin, blin):
    raise NotImplementedError("write your pallas kernel here")


# speedup vs baseline: 7.9978x; 7.9978x over previous
import functools

import jax
import jax.numpy as jnp
from jax import lax
from jax.experimental import pallas as pl
from jax.experimental.pallas import tpu as pltpu
from jax.experimental.pallas import tpu_sc as plsc

_N = 10000
_D = 128
_K = 128
_NC = 2
_NS = 16
_NW = _NC * _NS
_EPS = 1e-16
_BM = 1000


def _mesh():
    return plsc.VectorSubcoreMesh(
        core_axis_name="c", subcore_axis_name="s",
        num_cores=_NC, num_subcores=_NS)


def _elu(v):
    return jnp.where(v > 0, v, jnp.exp(v) - 1.0)


def _mm_dual(xp, wa, wb):
    npad, kdim = xp.shape
    ca, cb = wa.shape[1], wb.shape[1]

    def body(x_ref, wa_ref, wb_ref, oa_ref, ob_ref):
        x = x_ref[...]
        oa_ref[...] = jnp.dot(x, wa_ref[...], preferred_element_type=jnp.float32)
        ob_ref[...] = jnp.dot(x, wb_ref[...], preferred_element_type=jnp.float32)

    return pl.pallas_call(
        body,
        grid=(npad // _BM,),
        in_specs=[pl.BlockSpec((_BM, kdim), lambda i: (i, 0)),
                  pl.BlockSpec((kdim, ca), lambda i: (0, 0)),
                  pl.BlockSpec((kdim, cb), lambda i: (0, 0))],
        out_specs=[pl.BlockSpec((_BM, ca), lambda i: (i, 0)),
                   pl.BlockSpec((_BM, cb), lambda i: (i, 0))],
        out_shape=[jax.ShapeDtypeStruct((npad, ca), jnp.float32),
                   jax.ShapeDtypeStruct((npad, cb), jnp.float32)],
    )(xp, wa, wb)


def _norm(m0_ref, m1_ref, den_ref, nh, kdim):
    d = jnp.sum(den_ref[...], axis=0) + _EPS
    if nh > 1:
        expand = jnp.repeat(jnp.eye(nh, dtype=jnp.float32),
                            kdim // nh, axis=1)
        dexp = jnp.dot(d[:, :nh], expand, preferred_element_type=jnp.float32)
    else:
        dexp = d[:, :1]
    return (m0_ref[...] + m1_ref[...]) / dexp


def _act_mm_dual(m0, m1, den, nh, bias, wa, wb):
    npad, kdim = m0.shape
    ca, cb = wa.shape[1], wb.shape[1]

    def body(m0_ref, m1_ref, d_ref, b_ref, wa_ref, wb_ref, o_ref):
        h = _elu(_norm(m0_ref, m1_ref, d_ref, nh, kdim) + b_ref[...])
        o_ref[:, :ca] = jnp.dot(h, wa_ref[...], preferred_element_type=jnp.float32)
        o_ref[:, ca:] = jnp.dot(h, wb_ref[...], preferred_element_type=jnp.float32)

    return pl.pallas_call(
        body,
        grid=(npad // _BM,),
        in_specs=[pl.BlockSpec((_BM, kdim), lambda i: (i, 0)),
                  pl.BlockSpec((_BM, kdim), lambda i: (i, 0)),
                  pl.BlockSpec((_NW, _BM, 8), lambda i: (0, i, 0)),
                  pl.BlockSpec((1, kdim), lambda i: (0, 0)),
                  pl.BlockSpec((kdim, ca), lambda i: (0, 0)),
                  pl.BlockSpec((kdim, cb), lambda i: (0, 0))],
        out_specs=pl.BlockSpec((_BM, ca + cb), lambda i: (i, 0)),
        out_shape=jax.ShapeDtypeStruct((npad, ca + cb), jnp.float32),
    )(m0, m1, den, bias, wa, wb)


def _final_head(m0, m1, den, bias, wlin, blin):
    npad, kdim = m0.shape
    co = wlin.shape[1]

    def body(m0_ref, m1_ref, d_ref, b_ref, w_ref, bl_ref, o_ref):
        h = _elu(_norm(m0_ref, m1_ref, d_ref, 1, kdim) + b_ref[...])
        o = jnp.dot(h, w_ref[...], preferred_element_type=jnp.float32)
        o_ref[...] = _elu(o + bl_ref[...])

    return pl.pallas_call(
        body,
        grid=(npad // _BM,),
        in_specs=[pl.BlockSpec((_BM, kdim), lambda i: (i, 0)),
                  pl.BlockSpec((_BM, kdim), lambda i: (i, 0)),
                  pl.BlockSpec((_NW, _BM, 8), lambda i: (0, i, 0)),
                  pl.BlockSpec((1, kdim), lambda i: (0, 0)),
                  pl.BlockSpec((kdim, co), lambda i: (0, 0)),
                  pl.BlockSpec((1, co), lambda i: (0, 0))],
        out_specs=pl.BlockSpec((_BM, co), lambda i: (i, 0)),
        out_shape=jax.ShapeDtypeStruct((npad, co), jnp.float32),
    )(m0, m1, den, bias, wlin, blin)


def _make_attn(cht, ch, nh, ep, ne, loff, roff):
    epw = ep // _NW
    nchunks = epw // _K
    gp = _K // 16
    cpc = ch // nh

    @functools.partial(
        pl.kernel,
        out_type=(
            jax.ShapeDtypeStruct((ep * 8,), jnp.float32),
            jax.ShapeDtypeStruct((_NW * _N * 8,), jnp.float32),
        ),
        mesh=_mesh(),
        compiler_params=pltpu.CompilerParams(needs_layout_passes=False),
        scratch_types=[
            pltpu.VMEM((_K,), jnp.int32),
            pltpu.VMEM((_K,), jnp.int32),
            pltpu.VMEM((_K, cht), jnp.float32),
            pltpu.VMEM((_K, cht), jnp.float32),
            pltpu.VMEM((_K * 8,), jnp.float32),
            pltpu.VMEM((ch,), jnp.float32),
            pltpu.VMEM((_N * 8,), jnp.float32),
            pltpu.SemaphoreType.DMA,
            pltpu.SemaphoreType.DMA,
        ],
    )
    def attn(xl_hbm, xr_hbm, src_hbm, dst_hbm, att_hbm,
             ea_hbm, den_hbm,
             idxs, idxd, xlb, xrb, eabf, attv, denb, sem0, sem1):
        cid = lax.axis_index("c")
        sid = lax.axis_index("s")
        wid = sid * _NC + cid
        lanes = lax.iota(jnp.int32, 16)
        zero16 = jnp.zeros((16,), jnp.float32)
        hmask = lanes < 8
        for i in range(_K * 8 // 16):
            eabf[pl.ds(i * 16, 16)] = zero16

        def zpriv(i, carry):
            denb[pl.ds(i * 16, 16)] = zero16
            return carry
        lax.fori_loop(0, _N * 8 // 16, zpriv, 0)
        pltpu.sync_copy(att_hbm, attv)

        base = wid * epw
        att_vecs = [attv[pl.ds(j * 16, 16)] for j in range(ch // 16)]

        def chunk(i, carry):
            eb = pl.multiple_of(base + i * _K, _K)
            pltpu.sync_copy(src_hbm.at[pl.ds(eb, _K)], idxs)
            pltpu.sync_copy(dst_hbm.at[pl.ds(eb, _K)], idxd)
            ga = pltpu.async_copy(xl_hbm.at[idxs], xlb, sem0)
            gb = pltpu.async_copy(xr_hbm.at[idxd], xrb, sem1)
            ga.wait()
            gb.wait()

            def group(g, gc):
                rows = lanes + g * 16
                live = (rows + eb) < ne
                acc = [jnp.zeros((16,), jnp.float32) for _ in range(nh)]
                for c in range(ch):
                    colc = jnp.full((16,), c, jnp.int32)
                    vl = plsc.load_gather(xlb, [rows, colc + loff])
                    vr = plsc.load_gather(xrb, [rows, colc + roff])
                    s = vl + vr
                    lk = jnp.maximum(s, 0.2 * s)
                    ac = att_vecs[c // 16][c % 16]
                    acc[c // cpc] = acc[c // cpc] + ac * lk
                for h in range(nh):
                    ea = jnp.where(live, jnp.exp(acc[h]), 0.0)
                    plsc.store_scatter(eabf, [rows * 8 + h], ea)
                return gc
            lax.fori_loop(0, gp, group, 0)

            def dgroup(g, gc):
                dvec = idxd[pl.ds(g * 16, 16)]
                for e in range(16):
                    dst = dvec[e]
                    ev = plsc.load_gather(
                        eabf,
                        [jnp.where(hmask, (g * 16 + e) * 8 + lanes, 0)],
                        mask=hmask)
                    plsc.addupdate_scatter(
                        denb, [jnp.where(hmask, dst * 8 + lanes, 0)],
                        jnp.where(hmask, ev, 0.0), mask=hmask)
                return gc
            lax.fori_loop(0, gp, dgroup, 0)
            pltpu.sync_copy(eabf, ea_hbm.at[pl.ds(eb * 8, _K * 8)])
            return carry
        lax.fori_loop(0, nchunks, chunk, 0)
        pltpu.sync_copy(denb, den_hbm.at[pl.ds(wid * _N * 8, _N * 8)])

    return attn


def _make_msg(cht, ch, nh, ep, loff, pack):
    epw = ep // _NW
    nchunks = epw // _K
    gp = _K // 16
    cpc = ch // nh
    nr = _N // 2 if pack else _N
    ngrp = nr // 8
    gpt = (ngrp + _NS - 1) // _NS

    @functools.partial(
        pl.kernel,
        out_type=(
            jax.ShapeDtypeStruct((nr, 128), jnp.float32),
            jax.ShapeDtypeStruct((nr, 128), jnp.float32),
        ),
        mesh=_mesh(),
        compiler_params=pltpu.CompilerParams(needs_layout_passes=False),
        scratch_types=[
            pltpu.VMEM((_K,), jnp.int32),
            pltpu.VMEM((_K,), jnp.int32),
            pltpu.VMEM((_K, cht), jnp.float32),
            pltpu.VMEM((_K * 8,), jnp.float32),
            pltpu.VMEM((_K, 128), jnp.float32),
            pltpu.VMEM((8, 128), jnp.float32),
            pltpu.VMEM_SHARED((nr, 128), jnp.float32),
            pltpu.SemaphoreType.DMA,
        ],
    )
    def msg(xl_hbm, ea_hbm, src_hbm, dst_hbm,
            msg0_hbm, msg1_hbm,
            idxs, idxd, xlb, eabf, msgb, zb, acc_sh, sem0):
        cid = lax.axis_index("c")
        sid = lax.axis_index("s")
        wid = sid * _NC + cid
        lanes = lax.iota(jnp.int32, 16)
        zero16 = jnp.zeros((16,), jnp.float32)
        for i in range(8):
            for j in range(8):
                zb[i, pl.ds(j * 16, 16)] = zero16

        def zbody(j, carry):
            gid = sid + _NS * j

            @pl.when(gid < ngrp)
            def _():
                pltpu.sync_copy(
                    zb, acc_sh.at[pl.ds(pl.multiple_of(gid * 8, 8), 8)])
            return carry
        lax.fori_loop(0, gpt, zbody, 0)
        plsc.subcore_barrier()

        base = wid * epw

        def chunk(i, carry):
            eb = pl.multiple_of(base + i * _K, _K)
            pltpu.sync_copy(src_hbm.at[pl.ds(eb, _K)], idxs)
            pltpu.sync_copy(dst_hbm.at[pl.ds(eb, _K)], idxd)
            pltpu.async_copy(xl_hbm.at[idxs], xlb, sem0).wait()
            pltpu.sync_copy(ea_hbm.at[pl.ds(eb * 8, _K * 8)], eabf)

            def group(g, gc):
                rows = lanes + g * 16
                if pack:
                    dv = plsc.load_gather(idxd, [rows])
                    half = dv // 2
                    idxd[pl.ds(g * 16, 16)] = half
                    coff = (dv - half * 2) * 64
                for h in range(nh):
                    a = plsc.load_gather(eabf, [rows * 8 + h])
                    for c1 in range(cpc):
                        c = h * cpc + c1
                        colc = jnp.full((16,), c, jnp.int32)
                        xv = plsc.load_gather(xlb, [rows, colc + loff])
                        if pack:
                            plsc.store_scatter(
                                msgb, [rows, colc + coff], xv * a)
                            plsc.store_scatter(
                                msgb, [rows, colc + (64 - coff)], zero16)
                        else:
                            plsc.store_scatter(msgb, [rows, colc], xv * a)
                return gc
            lax.fori_loop(0, gp, group, 0)
            pltpu.sync_copy(msgb, acc_sh.at[idxd], add=True)
            return carry
        lax.fori_loop(0, nchunks, chunk, 0)
        plsc.subcore_barrier()

        def dump(j, carry):
            gid = sid + _NS * j

            @pl.when(gid < ngrp)
            def _():
                off = pl.multiple_of(gid * 8, 8)

                @pl.when(cid == 0)
                def _():
                    pltpu.sync_copy(acc_sh.at[pl.ds(off, 8)],
                                    msg0_hbm.at[pl.ds(off, 8)])

                @pl.when(cid == 1)
                def _():
                    pltpu.sync_copy(acc_sh.at[pl.ds(off, 8)],
                                    msg1_hbm.at[pl.ds(off, 8)])
            return carry
        lax.fori_loop(0, gpt, dump, 0)

    return msg


def kernel(x, edge_index, Wl1, Wr1, att1, b1, Wl2, Wr2, att2, b2, Wlin, blin):
    e = edge_index.shape[1]
    ne = e + _N
    ep = ((ne + _NW * _K - 1) // (_NW * _K)) * (_NW * _K)
    pad = ep - ne
    loops = jnp.arange(_N, dtype=jnp.int32)
    fill = jnp.zeros((pad,), jnp.int32)
    src = jnp.concatenate([edge_index[0].astype(jnp.int32), loops, fill])
    dst = jnp.concatenate([edge_index[1].astype(jnp.int32), loops, fill])

    xl1, xr1 = _mm_dual(x, Wl1, Wr1)
    ea1, den1 = _make_attn(128, 128, 8, ep, ne, 0, 0)(
        xl1, xr1, src, dst, att1.reshape(-1))
    m1a, m1b = _make_msg(128, 128, 8, ep, 0, False)(xl1, ea1, src, dst)

    den1r = den1.reshape(_NW, _N, 8)
    packed2 = _act_mm_dual(m1a, m1b, den1r, 8,
                           b1.reshape(1, -1), Wl2, Wr2)
    ea2, den2 = _make_attn(128, 64, 1, ep, ne, 0, 64)(
        packed2, packed2, src, dst, att2.reshape(-1))
    m2a, m2b = _make_msg(128, 64, 1, ep, 0, True)(packed2, ea2, src, dst)

    den2r = den2.reshape(_NW, _N, 8)
    out = _final_head(m2a.reshape(_N, 64), m2b.reshape(_N, 64), den2r,
                      b2.reshape(1, -1), Wlin, blin.reshape(1, -1))
    return out
